# baseline (speedup 1.0000x reference)
import functools

import jax
import jax.numpy as jnp
from jax import lax
from jax.experimental import pallas as pl
from jax.experimental.pallas import tpu as pltpu
from jax.experimental.pallas import tpu_sc as plsc

B = 16384
F = 32
N_ROWS = 1000000
NC = 2
NS = 16
L = 16
NW = NC * NS
BPW = B // NW
CHUNKS = BPW // L
EPW = F * BPW
CSTR = 999936
NAUX = N_ROWS - CSTR
AUXO = F * CSTR
FLAT = AUXO + NAUX * F

_mesh = plsc.VectorSubcoreMesh(core_axis_name="c", subcore_axis_name="s")


def _detile_body(ut_hbm, it_hbm, ua_hbm, ia_hbm, uo_hbm, io_hbm, sem):
    copies = []
    for f in range(F):
        for src, dst in ((ut_hbm, uo_hbm), (it_hbm, io_hbm)):
            copies.append(pltpu.make_async_copy(
                src.at[f, pl.ds(0, CSTR)],
                dst.at[pl.ds(f * CSTR, CSTR)], sem))
    copies.append(pltpu.make_async_copy(
        ua_hbm, uo_hbm.at[pl.ds(AUXO, NAUX * F)], sem))
    copies.append(pltpu.make_async_copy(
        ia_hbm, io_hbm.at[pl.ds(AUXO, NAUX * F)], sem))
    for c in copies:
        c.start()
    for c in copies:
        c.wait()


_detile = pl.pallas_call(
    _detile_body,
    in_specs=[pl.BlockSpec(memory_space=pl.ANY)] * 4,
    out_specs=[pl.BlockSpec(memory_space=pl.ANY)] * 2,
    out_shape=[
        jax.ShapeDtypeStruct((FLAT,), jnp.float32),
        jax.ShapeDtypeStruct((FLAT,), jnp.float32),
    ],
    scratch_shapes=[pltpu.SemaphoreType.DMA],
)


@functools.partial(
    pl.kernel,
    mesh=_mesh,
    out_type=jax.ShapeDtypeStruct((B,), jnp.float32),
    compiler_params=pltpu.CompilerParams(
        needs_layout_passes=False, use_tc_tiling_on_sc=False),
    scratch_types=[
        pltpu.VMEM((EPW,), jnp.int32),
        pltpu.VMEM((EPW,), jnp.int32),
        pltpu.VMEM((BPW,), jnp.int32),
        pltpu.VMEM((BPW,), jnp.int32),
        pltpu.VMEM((EPW,), jnp.float32),
        pltpu.VMEM((EPW,), jnp.float32),
        pltpu.VMEM((BPW,), jnp.float32),
        pltpu.VMEM((BPW,), jnp.float32),
        pltpu.VMEM((L,), jnp.float32),
        pltpu.VMEM((BPW,), jnp.float32),
        pltpu.SemaphoreType.DMA,
    ],
)
def _mf_kernel(uidx_hbm, iidx_hbm, uid_hbm, iid_hbm, uf_hbm, if_hbm,
               ub_hbm, ib_hbm, gb_hbm,
               out_hbm,
               uidx_v, iidx_v, uid_v, iid_v, u_data, i_data, ub_v, ib_v,
               gb_v, out_v, sem):
    wid = lax.axis_index("s") * NC + lax.axis_index("c")
    base = wid * BPW

    pltpu.sync_copy(uidx_hbm.at[wid], uidx_v)
    pltpu.sync_copy(iidx_hbm.at[wid], iidx_v)
    pltpu.sync_copy(uid_hbm.at[pl.ds(base, BPW)], uid_v)
    pltpu.sync_copy(iid_hbm.at[pl.ds(base, BPW)], iid_v)

    cu = pltpu.async_copy(uf_hbm.at[uidx_v], u_data, sem)
    ci = pltpu.async_copy(if_hbm.at[iidx_v], i_data, sem)
    cub = pltpu.async_copy(ub_hbm.at[uid_v], ub_v, sem)
    cib = pltpu.async_copy(ib_hbm.at[iid_v], ib_v, sem)
    pltpu.sync_copy(gb_hbm.at[...], gb_v)
    cu.wait()
    ci.wait()
    cub.wait()
    cib.wait()

    gb = gb_v[...]
    for c in range(CHUNKS):
        acc = ub_v[pl.ds(c * L, L)] + ib_v[pl.ds(c * L, L)] + gb
        for f in range(F):
            o = f * BPW + c * L
            acc = acc + u_data[pl.ds(o, L)] * i_data[pl.ds(o, L)]
        out_v[pl.ds(c * L, L)] = acc

    pltpu.sync_copy(out_v, out_hbm.at[pl.ds(base, BPW)])


def kernel(user_ids, item_ids, user_factors, item_factors, user_bias,
           item_bias, global_bias):
    uid = user_ids.astype(jnp.int32)
    iid = item_ids.astype(jnp.int32)
    foffs = (jnp.arange(F, dtype=jnp.int32) * CSTR).reshape(1, F, 1)
    frng = jnp.arange(F, dtype=jnp.int32).reshape(1, F, 1)

    def mkidx(ids):
        i3 = ids.reshape(NW, 1, BPW)
        main = i3 + foffs
        aux = AUXO + (i3 - CSTR) * F + frng
        return jnp.where(i3 < CSTR, main, aux).reshape(NW, EPW)

    uidx = mkidx(uid)
    iidx = mkidx(iid)
    uaux = user_factors[CSTR:, :].reshape(-1)
    iaux = item_factors[CSTR:, :].reshape(-1)
    uf, itf = _detile(user_factors.T, item_factors.T, uaux, iaux)
    ub = user_bias.reshape(-1)
    ib = item_bias.reshape(-1)
    gb = jnp.broadcast_to(global_bias.astype(jnp.float32), (L,))
    return _mf_kernel(uidx, iidx, uid, iid, uf, itf, ub, ib, gb)


# speedup vs baseline: 21.2746x; 21.2746x over previous
import functools

import jax
import jax.numpy as jnp
from jax import lax
from jax.experimental import pallas as pl
from jax.experimental.pallas import tpu as pltpu
from jax.experimental.pallas import tpu_sc as plsc

B = 16384
F = 32
N_ROWS = 1000000
NC = 2
NS = 16
L = 16
NW = NC * NS
BPW = B // NW
CHUNKS = BPW // L
EPW = F * BPW
CSTR = 999936
NAUX = N_ROWS - CSTR
AUXO = F * CSTR
FLAT = AUXO + NAUX * F

_mesh = plsc.VectorSubcoreMesh(core_axis_name="c", subcore_axis_name="s")


CHW = 249984
NCH = CSTR // CHW
NSLOT = 2


def _detile_body(ut_hbm, it_hbm, ua_hbm, ia_hbm, uo_hbm, io_hbm,
                 vbuf, rsem, wsem):
    steps = []
    for src, dst in ((ut_hbm, uo_hbm), (it_hbm, io_hbm)):
        for g in range(F // 8):
            for cc in range(NCH):
                steps.append((src, dst, g, cc))

    def read(i, slot):
        src, dst, g, cc = steps[i]
        pltpu.make_async_copy(
            src.at[pl.ds(8 * g, 8), pl.ds(cc * CHW, CHW)],
            vbuf.at[slot], rsem).start()

    read(0, 0)
    for i in range(len(steps)):
        slot = i % NSLOT
        src, dst, g, cc = steps[i]
        pltpu.make_async_copy(
            src.at[pl.ds(8 * g, 8), pl.ds(cc * CHW, CHW)],
            vbuf.at[slot], rsem).wait()
        if i + 1 < len(steps):
            read(i + 1, (i + 1) % NSLOT)
        writes = []
        for k in range(8):
            writes.append(pltpu.make_async_copy(
                vbuf.at[slot, k],
                dst.at[pl.ds((8 * g + k) * CSTR + cc * CHW, CHW)], wsem))
        for w in writes:
            w.start()
        for w in writes:
            w.wait()
    ca = pltpu.make_async_copy(ua_hbm, uo_hbm.at[pl.ds(AUXO, NAUX * F)],
                               wsem)
    cb = pltpu.make_async_copy(ia_hbm, io_hbm.at[pl.ds(AUXO, NAUX * F)],
                               wsem)
    ca.start()
    cb.start()
    ca.wait()
    cb.wait()


_detile = pl.pallas_call(
    _detile_body,
    in_specs=[pl.BlockSpec(memory_space=pl.ANY)] * 4,
    out_specs=[pl.BlockSpec(memory_space=pl.ANY)] * 2,
    out_shape=[
        jax.ShapeDtypeStruct((FLAT,), jnp.float32),
        jax.ShapeDtypeStruct((FLAT,), jnp.float32),
    ],
    scratch_shapes=[
        pltpu.VMEM((NSLOT, 8, CHW), jnp.float32),
        pltpu.SemaphoreType.DMA,
        pltpu.SemaphoreType.DMA,
    ],
)


@functools.partial(
    pl.kernel,
    mesh=_mesh,
    out_type=jax.ShapeDtypeStruct((B,), jnp.float32),
    compiler_params=pltpu.CompilerParams(
        needs_layout_passes=False, use_tc_tiling_on_sc=False),
    scratch_types=[
        pltpu.VMEM((EPW,), jnp.int32),
        pltpu.VMEM((EPW,), jnp.int32),
        pltpu.VMEM((BPW,), jnp.int32),
        pltpu.VMEM((BPW,), jnp.int32),
        pltpu.VMEM((EPW,), jnp.float32),
        pltpu.VMEM((EPW,), jnp.float32),
        pltpu.VMEM((BPW,), jnp.float32),
        pltpu.VMEM((BPW,), jnp.float32),
        pltpu.VMEM((L,), jnp.float32),
        pltpu.VMEM((BPW,), jnp.float32),
        pltpu.SemaphoreType.DMA,
    ],
)
def _mf_kernel(uidx_hbm, iidx_hbm, uid_hbm, iid_hbm, uf_hbm, if_hbm,
               ub_hbm, ib_hbm, gb_hbm,
               out_hbm,
               uidx_v, iidx_v, uid_v, iid_v, u_data, i_data, ub_v, ib_v,
               gb_v, out_v, sem):
    wid = lax.axis_index("s") * NC + lax.axis_index("c")
    base = wid * BPW

    pltpu.sync_copy(uidx_hbm.at[wid], uidx_v)
    pltpu.sync_copy(iidx_hbm.at[wid], iidx_v)
    pltpu.sync_copy(uid_hbm.at[pl.ds(base, BPW)], uid_v)
    pltpu.sync_copy(iid_hbm.at[pl.ds(base, BPW)], iid_v)

    cu = pltpu.async_copy(uf_hbm.at[uidx_v], u_data, sem)
    ci = pltpu.async_copy(if_hbm.at[iidx_v], i_data, sem)
    cub = pltpu.async_copy(ub_hbm.at[uid_v], ub_v, sem)
    cib = pltpu.async_copy(ib_hbm.at[iid_v], ib_v, sem)
    pltpu.sync_copy(gb_hbm.at[...], gb_v)
    cu.wait()
    ci.wait()
    cub.wait()
    cib.wait()

    gb = gb_v[...]
    for c in range(CHUNKS):
        acc = ub_v[pl.ds(c * L, L)] + ib_v[pl.ds(c * L, L)] + gb
        for f in range(F):
            o = f * BPW + c * L
            acc = acc + u_data[pl.ds(o, L)] * i_data[pl.ds(o, L)]
        out_v[pl.ds(c * L, L)] = acc

    pltpu.sync_copy(out_v, out_hbm.at[pl.ds(base, BPW)])


def kernel(user_ids, item_ids, user_factors, item_factors, user_bias,
           item_bias, global_bias):
    uid = user_ids.astype(jnp.int32)
    iid = item_ids.astype(jnp.int32)
    foffs = (jnp.arange(F, dtype=jnp.int32) * CSTR).reshape(1, F, 1)
    frng = jnp.arange(F, dtype=jnp.int32).reshape(1, F, 1)

    def mkidx(ids):
        i3 = ids.reshape(NW, 1, BPW)
        main = i3 + foffs
        aux = AUXO + (i3 - CSTR) * F + frng
        return jnp.where(i3 < CSTR, main, aux).reshape(NW, EPW)

    uidx = mkidx(uid)
    iidx = mkidx(iid)
    uaux = user_factors[CSTR:, :].reshape(-1)
    iaux = item_factors[CSTR:, :].reshape(-1)
    uf, itf = _detile(user_factors.T, item_factors.T, uaux, iaux)
    ub = user_bias.reshape(-1)
    ib = item_bias.reshape(-1)
    gb = jnp.broadcast_to(global_bias.astype(jnp.float32), (L,))
    return _mf_kernel(uidx, iidx, uid, iid, uf, itf, ub, ib, gb)


# speedup vs baseline: 23.7154x; 1.1147x over previous
import functools

import jax
import jax.numpy as jnp
from jax import lax
from jax.experimental import pallas as pl
from jax.experimental.pallas import tpu as pltpu
from jax.experimental.pallas import tpu_sc as plsc

B = 16384
F = 32
N_ROWS = 1000000
NC = 2
NS = 16
L = 16
NW = NC * NS
BPW = B // NW
CHUNKS = BPW // L
EPW = F * BPW
CSTR = 999936
NAUX = N_ROWS - CSTR
AUXO = F * CSTR
FLAT = AUXO + NAUX * F

_mesh = plsc.VectorSubcoreMesh(core_axis_name="c", subcore_axis_name="s")


CHW = 83328
NCH = CSTR // CHW
NSLOT = 6
RAH = 3


def _detile_body(ut_hbm, it_hbm, ua_hbm, ia_hbm, uo_hbm, io_hbm,
                 vbuf, rsem, wsem):
    steps = []
    for src, dst in ((ut_hbm, uo_hbm), (it_hbm, io_hbm)):
        for g in range(F // 8):
            for cc in range(NCH):
                steps.append((src, dst, g, cc))
    n = len(steps)

    def rdesc(i):
        src, dst, g, cc = steps[i]
        return pltpu.make_async_copy(
            src.at[pl.ds(8 * g, 8), pl.ds(cc * CHW, CHW)],
            vbuf.at[i % NSLOT], rsem.at[i % NSLOT])

    def wdesc(i, k):
        src, dst, g, cc = steps[i]
        return pltpu.make_async_copy(
            vbuf.at[i % NSLOT, k],
            dst.at[pl.ds((8 * g + k) * CSTR + cc * CHW, CHW)],
            wsem.at[i % NSLOT])

    for i in range(RAH):
        rdesc(i).start()
    for i in range(n):
        rdesc(i).wait()
        for k in range(8):
            wdesc(i, k).start()
        j = i + RAH
        if j < n:
            jj = j - NSLOT
            if jj >= 0:
                for k in range(8):
                    wdesc(jj, k).wait()
            rdesc(j).start()
    for jj in range(max(0, n - NSLOT), n):
        for k in range(8):
            wdesc(jj, k).wait()
    ca = pltpu.make_async_copy(ua_hbm, uo_hbm.at[pl.ds(AUXO, NAUX * F)],
                               rsem.at[0])
    cb = pltpu.make_async_copy(ia_hbm, io_hbm.at[pl.ds(AUXO, NAUX * F)],
                               rsem.at[1])
    ca.start()
    cb.start()
    ca.wait()
    cb.wait()


_detile = pl.pallas_call(
    _detile_body,
    in_specs=[pl.BlockSpec(memory_space=pl.ANY)] * 4,
    out_specs=[pl.BlockSpec(memory_space=pl.ANY)] * 2,
    out_shape=[
        jax.ShapeDtypeStruct((FLAT,), jnp.float32),
        jax.ShapeDtypeStruct((FLAT,), jnp.float32),
    ],
    scratch_shapes=[
        pltpu.VMEM((NSLOT, 8, CHW), jnp.float32),
        pltpu.SemaphoreType.DMA((NSLOT,)),
        pltpu.SemaphoreType.DMA((NSLOT,)),
    ],
)


@functools.partial(
    pl.kernel,
    mesh=_mesh,
    out_type=jax.ShapeDtypeStruct((B,), jnp.float32),
    compiler_params=pltpu.CompilerParams(
        needs_layout_passes=False, use_tc_tiling_on_sc=False),
    scratch_types=[
        pltpu.VMEM((EPW,), jnp.int32),
        pltpu.VMEM((EPW,), jnp.int32),
        pltpu.VMEM((BPW,), jnp.int32),
        pltpu.VMEM((BPW,), jnp.int32),
        pltpu.VMEM((EPW,), jnp.float32),
        pltpu.VMEM((EPW,), jnp.float32),
        pltpu.VMEM((BPW,), jnp.float32),
        pltpu.VMEM((BPW,), jnp.float32),
        pltpu.VMEM((L,), jnp.float32),
        pltpu.VMEM((BPW,), jnp.float32),
        pltpu.SemaphoreType.DMA,
    ],
)
def _mf_kernel(uidx_hbm, iidx_hbm, uid_hbm, iid_hbm, uf_hbm, if_hbm,
               ub_hbm, ib_hbm, gb_hbm,
               out_hbm,
               uidx_v, iidx_v, uid_v, iid_v, u_data, i_data, ub_v, ib_v,
               gb_v, out_v, sem):
    wid = lax.axis_index("s") * NC + lax.axis_index("c")
    base = wid * BPW

    pltpu.sync_copy(uidx_hbm.at[wid], uidx_v)
    pltpu.sync_copy(iidx_hbm.at[wid], iidx_v)
    pltpu.sync_copy(uid_hbm.at[pl.ds(base, BPW)], uid_v)
    pltpu.sync_copy(iid_hbm.at[pl.ds(base, BPW)], iid_v)

    cu = pltpu.async_copy(uf_hbm.at[uidx_v], u_data, sem)
    ci = pltpu.async_copy(if_hbm.at[iidx_v], i_data, sem)
    cub = pltpu.async_copy(ub_hbm.at[uid_v], ub_v, sem)
    cib = pltpu.async_copy(ib_hbm.at[iid_v], ib_v, sem)
    pltpu.sync_copy(gb_hbm.at[...], gb_v)
    cu.wait()
    ci.wait()
    cub.wait()
    cib.wait()

    gb = gb_v[...]
    for c in range(CHUNKS):
        acc = ub_v[pl.ds(c * L, L)] + ib_v[pl.ds(c * L, L)] + gb
        for f in range(F):
            o = f * BPW + c * L
            acc = acc + u_data[pl.ds(o, L)] * i_data[pl.ds(o, L)]
        out_v[pl.ds(c * L, L)] = acc

    pltpu.sync_copy(out_v, out_hbm.at[pl.ds(base, BPW)])


def kernel(user_ids, item_ids, user_factors, item_factors, user_bias,
           item_bias, global_bias):
    uid = user_ids.astype(jnp.int32)
    iid = item_ids.astype(jnp.int32)
    foffs = (jnp.arange(F, dtype=jnp.int32) * CSTR).reshape(1, F, 1)
    frng = jnp.arange(F, dtype=jnp.int32).reshape(1, F, 1)

    def mkidx(ids):
        i3 = ids.reshape(NW, 1, BPW)
        main = i3 + foffs
        aux = AUXO + (i3 - CSTR) * F + frng
        return jnp.where(i3 < CSTR, main, aux).reshape(NW, EPW)

    uidx = mkidx(uid)
    iidx = mkidx(iid)
    uaux = user_factors[CSTR:, :].reshape(-1)
    iaux = item_factors[CSTR:, :].reshape(-1)
    uf, itf = _detile(user_factors.T, item_factors.T, uaux, iaux)
    ub = user_bias.reshape(-1)
    ib = item_bias.reshape(-1)
    gb = jnp.broadcast_to(global_bias.astype(jnp.float32), (L,))
    return _mf_kernel(uidx, iidx, uid, iid, uf, itf, ub, ib, gb)


# speedup vs baseline: 23.7775x; 1.0026x over previous
import functools

import jax
import jax.numpy as jnp
from jax import lax
from jax.experimental import pallas as pl
from jax.experimental.pallas import tpu as pltpu
from jax.experimental.pallas import tpu_sc as plsc

B = 16384
F = 32
N_ROWS = 1000000
NC = 2
NS = 16
L = 16
NW = NC * NS
BPW = B // NW
CHUNKS = BPW // L
EPW = F * BPW
CSTR = 999936
NAUX = N_ROWS - CSTR
AUXO = F * CSTR
FLAT = AUXO + NAUX * F

_mesh = plsc.VectorSubcoreMesh(core_axis_name="c", subcore_axis_name="s")


CHW = 166656
NCH = CSTR // CHW
NSLOT = 5
RAH = 3


def _detile_body(ut_hbm, it_hbm, ua_hbm, ia_hbm, uo_hbm, io_hbm,
                 vbuf, rsem, wsem):
    steps = []
    for src, dst in ((ut_hbm, uo_hbm), (it_hbm, io_hbm)):
        for g in range(F // 8):
            for cc in range(NCH):
                steps.append((src, dst, g, cc))
    n = len(steps)

    def rdesc(i):
        src, dst, g, cc = steps[i]
        return pltpu.make_async_copy(
            src.at[pl.ds(8 * g, 8), pl.ds(cc * CHW, CHW)],
            vbuf.at[i % NSLOT], rsem.at[i % NSLOT])

    def wdesc(i, k):
        src, dst, g, cc = steps[i]
        return pltpu.make_async_copy(
            vbuf.at[i % NSLOT, k],
            dst.at[pl.ds((8 * g + k) * CSTR + cc * CHW, CHW)],
            wsem.at[i % NSLOT])

    for i in range(RAH):
        rdesc(i).start()
    for i in range(n):
        rdesc(i).wait()
        for k in range(8):
            wdesc(i, k).start()
        j = i + RAH
        if j < n:
            jj = j - NSLOT
            if jj >= 0:
                for k in range(8):
                    wdesc(jj, k).wait()
            rdesc(j).start()
    for jj in range(max(0, n - NSLOT), n):
        for k in range(8):
            wdesc(jj, k).wait()
    ca = pltpu.make_async_copy(ua_hbm, uo_hbm.at[pl.ds(AUXO, NAUX * F)],
                               rsem.at[0])
    cb = pltpu.make_async_copy(ia_hbm, io_hbm.at[pl.ds(AUXO, NAUX * F)],
                               rsem.at[1])
    ca.start()
    cb.start()
    ca.wait()
    cb.wait()


_detile = pl.pallas_call(
    _detile_body,
    in_specs=[pl.BlockSpec(memory_space=pl.ANY)] * 4,
    out_specs=[pl.BlockSpec(memory_space=pl.ANY)] * 2,
    out_shape=[
        jax.ShapeDtypeStruct((FLAT,), jnp.float32),
        jax.ShapeDtypeStruct((FLAT,), jnp.float32),
    ],
    scratch_shapes=[
        pltpu.VMEM((NSLOT, 8, CHW), jnp.float32),
        pltpu.SemaphoreType.DMA((NSLOT,)),
        pltpu.SemaphoreType.DMA((NSLOT,)),
    ],
)


@functools.partial(
    pl.kernel,
    mesh=_mesh,
    out_type=jax.ShapeDtypeStruct((B,), jnp.float32),
    compiler_params=pltpu.CompilerParams(
        needs_layout_passes=False, use_tc_tiling_on_sc=False),
    scratch_types=[
        pltpu.VMEM((EPW,), jnp.int32),
        pltpu.VMEM((EPW,), jnp.int32),
        pltpu.VMEM((BPW,), jnp.int32),
        pltpu.VMEM((BPW,), jnp.int32),
        pltpu.VMEM((EPW,), jnp.float32),
        pltpu.VMEM((EPW,), jnp.float32),
        pltpu.VMEM((BPW,), jnp.float32),
        pltpu.VMEM((BPW,), jnp.float32),
        pltpu.VMEM((L,), jnp.float32),
        pltpu.VMEM((BPW,), jnp.float32),
        pltpu.SemaphoreType.DMA,
    ],
)
def _mf_kernel(uidx_hbm, iidx_hbm, uid_hbm, iid_hbm, uf_hbm, if_hbm,
               ub_hbm, ib_hbm, gb_hbm,
               out_hbm,
               uidx_v, iidx_v, uid_v, iid_v, u_data, i_data, ub_v, ib_v,
               gb_v, out_v, sem):
    wid = lax.axis_index("s") * NC + lax.axis_index("c")
    base = wid * BPW

    pltpu.sync_copy(uidx_hbm.at[wid], uidx_v)
    pltpu.sync_copy(iidx_hbm.at[wid], iidx_v)
    pltpu.sync_copy(uid_hbm.at[pl.ds(base, BPW)], uid_v)
    pltpu.sync_copy(iid_hbm.at[pl.ds(base, BPW)], iid_v)

    cu = pltpu.async_copy(uf_hbm.at[uidx_v], u_data, sem)
    ci = pltpu.async_copy(if_hbm.at[iidx_v], i_data, sem)
    cub = pltpu.async_copy(ub_hbm.at[uid_v], ub_v, sem)
    cib = pltpu.async_copy(ib_hbm.at[iid_v], ib_v, sem)
    pltpu.sync_copy(gb_hbm.at[...], gb_v)
    cu.wait()
    ci.wait()
    cub.wait()
    cib.wait()

    gb = gb_v[...]
    for c in range(CHUNKS):
        acc = ub_v[pl.ds(c * L, L)] + ib_v[pl.ds(c * L, L)] + gb
        for f in range(F):
            o = f * BPW + c * L
            acc = acc + u_data[pl.ds(o, L)] * i_data[pl.ds(o, L)]
        out_v[pl.ds(c * L, L)] = acc

    pltpu.sync_copy(out_v, out_hbm.at[pl.ds(base, BPW)])


def kernel(user_ids, item_ids, user_factors, item_factors, user_bias,
           item_bias, global_bias):
    uid = user_ids.astype(jnp.int32)
    iid = item_ids.astype(jnp.int32)
    foffs = (jnp.arange(F, dtype=jnp.int32) * CSTR).reshape(1, F, 1)
    frng = jnp.arange(F, dtype=jnp.int32).reshape(1, F, 1)

    def mkidx(ids):
        i3 = ids.reshape(NW, 1, BPW)
        main = i3 + foffs
        aux = AUXO + (i3 - CSTR) * F + frng
        return jnp.where(i3 < CSTR, main, aux).reshape(NW, EPW)

    uidx = mkidx(uid)
    iidx = mkidx(iid)
    uaux = user_factors[CSTR:, :].reshape(-1)
    iaux = item_factors[CSTR:, :].reshape(-1)
    uf, itf = _detile(user_factors.T, item_factors.T, uaux, iaux)
    ub = user_bias.reshape(-1)
    ib = item_bias.reshape(-1)
    gb = jnp.broadcast_to(global_bias.astype(jnp.float32), (L,))
    return _mf_kernel(uidx, iidx, uid, iid, uf, itf, ub, ib, gb)


# speedup vs baseline: 25.0518x; 1.0536x over previous
import functools

import jax
import jax.numpy as jnp
from jax import lax
from jax.experimental import pallas as pl
from jax.experimental.pallas import tpu as pltpu
from jax.experimental.pallas import tpu_sc as plsc

B = 16384
F = 32
N_ROWS = 1000000
NC = 2
NS = 16
L = 16
NW = NC * NS
BPW = B // NW
CHUNKS = BPW // L
EPW = F * BPW
CSTR = 999936
NAUX = N_ROWS - CSTR
AUXO = F * CSTR
FLAT = AUXO + NAUX * F

_mesh = plsc.VectorSubcoreMesh(core_axis_name="c", subcore_axis_name="s")


CHW = 166656
NCH = CSTR // CHW
NSLOT = 5
RAH = 3


def _detile_body(ut_hbm, ua_hbm, uo_hbm, vbuf, rsem, wsem):
    steps = []
    for src, dst in ((ut_hbm, uo_hbm),):
        for g in range(F // 8):
            for cc in range(NCH):
                steps.append((src, dst, g, cc))
    n = len(steps)

    def rdesc(i):
        src, dst, g, cc = steps[i]
        return pltpu.make_async_copy(
            src.at[pl.ds(8 * g, 8), pl.ds(cc * CHW, CHW)],
            vbuf.at[i % NSLOT], rsem.at[i % NSLOT])

    def wdesc(i, k):
        src, dst, g, cc = steps[i]
        return pltpu.make_async_copy(
            vbuf.at[i % NSLOT, k],
            dst.at[pl.ds((8 * g + k) * CSTR + cc * CHW, CHW)],
            wsem.at[i % NSLOT])

    for i in range(RAH):
        rdesc(i).start()
    for i in range(n):
        rdesc(i).wait()
        for k in range(8):
            wdesc(i, k).start()
        j = i + RAH
        if j < n:
            jj = j - NSLOT
            if jj >= 0:
                for k in range(8):
                    wdesc(jj, k).wait()
            rdesc(j).start()
    for jj in range(max(0, n - NSLOT), n):
        for k in range(8):
            wdesc(jj, k).wait()
    ca = pltpu.make_async_copy(ua_hbm, uo_hbm.at[pl.ds(AUXO, NAUX * F)],
                               rsem.at[0])
    ca.start()
    ca.wait()


_detile = pl.pallas_call(
    _detile_body,
    in_specs=[pl.BlockSpec(memory_space=pl.ANY)] * 2,
    out_specs=pl.BlockSpec(memory_space=pl.ANY),
    out_shape=jax.ShapeDtypeStruct((FLAT,), jnp.float32),
    scratch_shapes=[
        pltpu.VMEM((NSLOT, 8, CHW), jnp.float32),
        pltpu.SemaphoreType.DMA((NSLOT,)),
        pltpu.SemaphoreType.DMA((NSLOT,)),
    ],
)


@functools.partial(
    pl.kernel,
    mesh=_mesh,
    out_type=jax.ShapeDtypeStruct((B * F,), jnp.float32),
    compiler_params=pltpu.CompilerParams(
        needs_layout_passes=False, use_tc_tiling_on_sc=False),
    scratch_types=[
        pltpu.VMEM((EPW,), jnp.int32),
        pltpu.VMEM((EPW,), jnp.float32),
        pltpu.SemaphoreType.DMA,
    ],
)
def _gather_u_kernel(uidx_hbm, uf_hbm, out_hbm, uidx_v, u_data, sem):
    wid = lax.axis_index("s") * NC + lax.axis_index("c")
    pltpu.sync_copy(uidx_hbm.at[wid], uidx_v)
    pltpu.async_copy(uf_hbm.at[uidx_v], u_data, sem).wait()
    pltpu.sync_copy(u_data, out_hbm.at[pl.ds(wid * EPW, EPW)])


@functools.partial(
    pl.kernel,
    mesh=_mesh,
    out_type=jax.ShapeDtypeStruct((B,), jnp.float32),
    compiler_params=pltpu.CompilerParams(
        needs_layout_passes=False, use_tc_tiling_on_sc=False),
    scratch_types=[
        pltpu.VMEM((EPW,), jnp.int32),
        pltpu.VMEM((BPW,), jnp.int32),
        pltpu.VMEM((BPW,), jnp.int32),
        pltpu.VMEM((EPW,), jnp.float32),
        pltpu.VMEM((EPW,), jnp.float32),
        pltpu.VMEM((BPW,), jnp.float32),
        pltpu.VMEM((BPW,), jnp.float32),
        pltpu.VMEM((L,), jnp.float32),
        pltpu.VMEM((BPW,), jnp.float32),
        pltpu.SemaphoreType.DMA,
    ],
)
def _mf_kernel(iidx_hbm, uid_hbm, iid_hbm, ug_hbm, if_hbm,
               ub_hbm, ib_hbm, gb_hbm,
               out_hbm,
               iidx_v, uid_v, iid_v, u_data, i_data, ub_v, ib_v,
               gb_v, out_v, sem):
    wid = lax.axis_index("s") * NC + lax.axis_index("c")
    base = wid * BPW

    pltpu.sync_copy(iidx_hbm.at[wid], iidx_v)
    pltpu.sync_copy(uid_hbm.at[pl.ds(base, BPW)], uid_v)
    pltpu.sync_copy(iid_hbm.at[pl.ds(base, BPW)], iid_v)

    ci = pltpu.async_copy(if_hbm.at[iidx_v], i_data, sem)
    cug = pltpu.async_copy(ug_hbm.at[pl.ds(wid * EPW, EPW)], u_data, sem)
    cub = pltpu.async_copy(ub_hbm.at[uid_v], ub_v, sem)
    cib = pltpu.async_copy(ib_hbm.at[iid_v], ib_v, sem)
    pltpu.sync_copy(gb_hbm.at[...], gb_v)
    ci.wait()
    cug.wait()
    cub.wait()
    cib.wait()

    gb = gb_v[...]
    for c in range(CHUNKS):
        acc = ub_v[pl.ds(c * L, L)] + ib_v[pl.ds(c * L, L)] + gb
        for f in range(F):
            o = f * BPW + c * L
            acc = acc + u_data[pl.ds(o, L)] * i_data[pl.ds(o, L)]
        out_v[pl.ds(c * L, L)] = acc

    pltpu.sync_copy(out_v, out_hbm.at[pl.ds(base, BPW)])


def kernel(user_ids, item_ids, user_factors, item_factors, user_bias,
           item_bias, global_bias):
    uid = user_ids.astype(jnp.int32)
    iid = item_ids.astype(jnp.int32)
    foffs = (jnp.arange(F, dtype=jnp.int32) * CSTR).reshape(1, F, 1)
    frng = jnp.arange(F, dtype=jnp.int32).reshape(1, F, 1)

    def mkidx(ids):
        i3 = ids.reshape(NW, 1, BPW)
        main = i3 + foffs
        aux = AUXO + (i3 - CSTR) * F + frng
        return jnp.where(i3 < CSTR, main, aux).reshape(NW, EPW)

    uidx = mkidx(uid)
    iidx = mkidx(iid)
    uaux = user_factors[CSTR:, :].reshape(-1)
    iaux = item_factors[CSTR:, :].reshape(-1)
    uf = _detile(user_factors.T, uaux)
    ug = _gather_u_kernel(uidx, uf)
    itf = _detile(item_factors.T, iaux)
    ub = user_bias.reshape(-1)
    ib = item_bias.reshape(-1)
    gb = jnp.broadcast_to(global_bias.astype(jnp.float32), (L,))
    return _mf_kernel(iidx, uid, iid, ug, itf, ub, ib, gb)
